# Initial kernel scaffold; baseline (speedup 1.0000x reference)
import jax
import jax.numpy as jnp
from jax.experimental import pallas as pl


def kernel(pos, params, z, edge_index, batch):
    raise NotImplementedError("write your pallas kernel here")


# speedup vs baseline: 6.8678x; 6.8678x over previous
import functools
import math

import jax
import jax.numpy as jnp
from jax import lax
from jax.experimental import pallas as pl
from jax.experimental.pallas import tpu as pltpu
from jax.experimental.pallas import tpu_sc as plsc

_N = 10000
_E = 320000
_G = 100
_EMBED = 128
_HEADS = 4
_HD = 32
_NR = 32
_CUTOFF = 6.0
_NDP = 512
_OD = 9

_NCORES = 2
_NSUB = 16
_NW = _NCORES * _NSUB
_EPW = _E // _NW
_BE = 80
_NBLK = _EPW // _BE
_NP = 10240
_NPT = _NP // _NSUB

_BLK = 2000
_EBLK = 8000


def _sc_mesh():
    return plsc.VectorSubcoreMesh(
        core_axis_name="c", subcore_axis_name="s",
        num_cores=_NCORES, num_subcores=_NSUB)


def _d2_body(posT_h, src_h, dst_h, out_h, pos_v, src_v, dst_v, out_v, sem):
    cid = lax.axis_index("c")
    sid = lax.axis_index("s")
    wid = cid * _NSUB + sid
    base = wid * _EPW
    pltpu.sync_copy(posT_h, pos_v)
    pltpu.sync_copy(src_h.at[pl.ds(base, _EPW)], src_v)
    pltpu.sync_copy(dst_h.at[pl.ds(base, _EPW)], dst_v)
    iota16 = lax.iota(jnp.int32, 16)
    zero = jnp.full((16,), 0, jnp.int32)
    one = jnp.full((16,), 1, jnp.int32)
    two = jnp.full((16,), 2, jnp.int32)

    def body(g, _):
        s_ids = src_v[pl.ds(g * 16, 16)]
        d_ids = dst_v[pl.ds(g * 16, 16)]
        dx = (plsc.load_gather(pos_v, [zero, s_ids])
              - plsc.load_gather(pos_v, [zero, d_ids]))
        dy = (plsc.load_gather(pos_v, [one, s_ids])
              - plsc.load_gather(pos_v, [one, d_ids]))
        dz = (plsc.load_gather(pos_v, [two, s_ids])
              - plsc.load_gather(pos_v, [two, d_ids]))
        out_v[pl.ds(g * 16, 16)] = dx * dx + dy * dy + dz * dz
        return 0

    lax.fori_loop(0, _EPW // 16, body, 0)
    pltpu.sync_copy(out_v, out_h.at[pl.ds(base, _EPW)])


def _edge_d2(posT, src, dst):
    kern = pl.kernel(
        _d2_body,
        out_type=jax.ShapeDtypeStruct((_E,), jnp.float32),
        mesh=_sc_mesh(),
        scratch_types=[
            pltpu.VMEM((3, _N), jnp.float32),
            pltpu.VMEM((_EPW,), jnp.int32),
            pltpu.VMEM((_EPW,), jnp.int32),
            pltpu.VMEM((_EPW,), jnp.float32),
            pltpu.SemaphoreType.DMA,
        ],
        compiler_params=pltpu.CompilerParams(needs_layout_passes=False, use_tc_tiling_on_sc=False),
    )
    return kern(posT, src, dst)


def _edge_body(xq_h, xk_h, xv_h, e_h, src_h, dst_h, out_m, out_s,
               src_v, dst_v, q_v, k_v, v_v, e_v, m_v, s_v, zb, zbs,
               msg_sh, s_sh, sem):
    cid = lax.axis_index("c")
    sid = lax.axis_index("s")
    zero16 = jnp.zeros((16,), jnp.float32)

    def z1(i, _):
        zb[i // 4, pl.ds((i % 4) * 16, 16)] = zero16
        return 0
    lax.fori_loop(0, 128 * 4, z1, 0)

    def z2(i, _):
        zbs[i, pl.ds(0, 16)] = zero16
        return 0
    lax.fori_loop(0, _NPT, z2, 0)

    def z3(i, _):
        s_v[i, pl.ds(0, 16)] = zero16
        return 0
    lax.fori_loop(0, _BE, z3, 0)

    for t in range(5):
        pltpu.sync_copy(zb, msg_sh.at[pl.ds(sid * _NPT + t * 128, 128)])
    pltpu.sync_copy(zbs, s_sh.at[pl.ds(sid * _NPT, _NPT)])
    plsc.subcore_barrier()

    inv_sqrt = 1.0 / math.sqrt(float(_HD))
    iota16 = lax.iota(jnp.int32, 16)
    base_w = sid * (_E // _NSUB)

    def block(blk, _):
        base = base_w + blk * _BE
        pltpu.sync_copy(src_h.at[pl.ds(base, _BE)], src_v)
        pltpu.sync_copy(dst_h.at[pl.ds(base, _BE)], dst_v)
        pltpu.async_copy(xq_h.at[cid].at[dst_v], q_v, sem).wait()
        pltpu.async_copy(xk_h.at[cid].at[src_v], k_v, sem).wait()
        pltpu.async_copy(xv_h.at[cid].at[src_v], v_v, sem).wait()
        pltpu.sync_copy(e_h.at[cid, pl.ds(base, _BE)], e_v)

        def group(g, _):
            eids = iota16 + g * 16
            for hh in range(2):
                def dotstep(j, acc):
                    col = jnp.full((16,), hh * _HD + j, jnp.int32)
                    qc = plsc.load_gather(q_v, [eids, col])
                    kc = plsc.load_gather(k_v, [eids, col])
                    ec = plsc.load_gather(e_v, [eids, col])
                    return acc + qc * (kc + ec)
                lg = lax.fori_loop(0, _HD, dotstep, zero16)
                w = jnp.exp(lg * inv_sqrt)
                lane = jnp.full((16,), 0, jnp.int32) + (cid * 2 + hh)
                plsc.store_scatter(s_v, [eids, lane], w)

                def mstep(j, _):
                    col = jnp.full((16,), hh * _HD + j, jnp.int32)
                    vc = plsc.load_gather(v_v, [eids, col])
                    ec = plsc.load_gather(e_v, [eids, col])
                    plsc.store_scatter(m_v, [eids, col], w * (vc + ec))
                    return 0
                lax.fori_loop(0, _HD, mstep, 0)
            return 0

        lax.fori_loop(0, _BE // 16, group, 0)
        pltpu.sync_copy(m_v, msg_sh.at[dst_v], add=True)
        pltpu.sync_copy(s_v, s_sh.at[dst_v], add=True)
        return 0

    lax.fori_loop(0, (_E // _NSUB) // _BE, block, 0)
    plsc.subcore_barrier()
    pltpu.sync_copy(msg_sh.at[pl.ds(sid * _NPT, _NPT)],
                    out_m.at[cid, pl.ds(sid * _NPT, _NPT)])
    pltpu.sync_copy(s_sh.at[pl.ds(sid * _NPT, _NPT)],
                    out_s.at[cid, pl.ds(sid * _NPT, _NPT)])


def _edge_attn(xq2, xk2, xv2, e2, src, dst):
    kern = pl.kernel(
        _edge_body,
        out_type=(jax.ShapeDtypeStruct((_NCORES, _NP, 64), jnp.float32),
                  jax.ShapeDtypeStruct((_NCORES, _NP, 16), jnp.float32)),
        mesh=_sc_mesh(),
        scratch_types=[
            pltpu.VMEM((_BE,), jnp.int32),
            pltpu.VMEM((_BE,), jnp.int32),
            pltpu.VMEM((_BE, 64), jnp.float32),
            pltpu.VMEM((_BE, 64), jnp.float32),
            pltpu.VMEM((_BE, 64), jnp.float32),
            pltpu.VMEM((_BE, 64), jnp.float32),
            pltpu.VMEM((_BE, 64), jnp.float32),
            pltpu.VMEM((_BE, 16), jnp.float32),
            pltpu.VMEM((128, 64), jnp.float32),
            pltpu.VMEM((_NPT, 16), jnp.float32),
            pltpu.VMEM_SHARED((_NP, 64), jnp.float32),
            pltpu.VMEM_SHARED((_NP, 16), jnp.float32),
            pltpu.SemaphoreType.DMA,
        ],
        compiler_params=pltpu.CompilerParams(needs_layout_passes=False, use_tc_tiling_on_sc=False),
    )
    return kern(xq2, xk2, xv2, e2, src, dst)


def _enc_body(z8, pos8, embp, wemb, wpos, binit, wq, wk, wv,
              x_o, xq_o, xk_o, xv_o):
    zcol = z8[:, 0:1].astype(jnp.int32)
    ioh = lax.broadcasted_iota(jnp.int32, (_BLK, 8), 1)
    oh = (ioh == zcol).astype(jnp.float32)
    femb = jnp.dot(oh, embp[...], preferred_element_type=jnp.float32)
    x = (jnp.dot(femb, wemb[...], preferred_element_type=jnp.float32)
         + jnp.dot(pos8[...], wpos[...], preferred_element_type=jnp.float32)
         + binit[...])
    x_o[...] = x
    xq = jnp.dot(x, wq[...], preferred_element_type=jnp.float32)
    xk = jnp.dot(x, wk[...], preferred_element_type=jnp.float32)
    xv = jnp.dot(x, wv[...], preferred_element_type=jnp.float32)
    xq_o[0] = xq[:, :64]
    xq_o[1] = xq[:, 64:]
    xk_o[0] = xk[:, :64]
    xk_o[1] = xk[:, 64:]
    xv_o[0] = xv[:, :64]
    xv_o[1] = xv[:, 64:]


def _encoder(z8, pos8, embp, wemb, wpos, binit, wq, wk, wv):
    full = lambda s: pl.BlockSpec(s, lambda i: (0, 0))
    out = jax.ShapeDtypeStruct((_N, _EMBED), jnp.float32)
    out2 = jax.ShapeDtypeStruct((2, _N, 64), jnp.float32)
    spec2 = pl.BlockSpec((2, _BLK, 64), lambda i: (0, i, 0))
    return pl.pallas_call(
        _enc_body,
        grid=(_N // _BLK,),
        in_specs=[
            pl.BlockSpec((_BLK, 8), lambda i: (i, 0)),
            pl.BlockSpec((_BLK, 8), lambda i: (i, 0)),
            full((8, 32)), full((32, _EMBED)), full((8, _EMBED)),
            full((1, _EMBED)), full((_EMBED, _EMBED)),
            full((_EMBED, _EMBED)), full((_EMBED, _EMBED)),
        ],
        out_specs=[pl.BlockSpec((_BLK, _EMBED), lambda i: (i, 0)),
                   spec2, spec2, spec2],
        out_shape=[out, out2, out2, out2],
    )(z8, pos8, embp, wemb, wpos, binit, wq, wk, wv)


def _e_body(d2, we, e_o):
    d = jnp.sqrt(d2[...] + 1e-12)
    width = _CUTOFF / _NR
    centers = lax.broadcasted_iota(jnp.int32, (1, _NR), 1).astype(
        jnp.float32) * (_CUTOFF / (_NR - 1))
    diff = d - centers
    rbf = jnp.exp(-(diff * diff) * (1.0 / (2.0 * width * width)))
    e = jnp.dot(rbf, we[...], preferred_element_type=jnp.float32)
    e_o[0] = e[:, :64]
    e_o[1] = e[:, 64:]


def _e_proj(d2c, we):
    return pl.pallas_call(
        _e_body,
        grid=(_E // _EBLK,),
        in_specs=[
            pl.BlockSpec((_EBLK, 1), lambda i: (i, 0)),
            pl.BlockSpec((_NR, _EMBED), lambda i: (0, 0)),
        ],
        out_specs=pl.BlockSpec((2, _EBLK, 64), lambda i: (0, i, 0)),
        out_shape=jax.ShapeDtypeStruct((2, _E, 64), jnp.float32),
    )(d2c, we)


def _merge_body(x, pm, ps, wfc, bfc, g, b, wq, wk, wv,
                x_o, xq_o, xk_o, xv_o):
    m = pm[...]
    msg = jnp.concatenate([m[0], m[1]], axis=1)
    s = ps[...]
    s4 = s[0] + s[1]
    den = jnp.concatenate(
        [jnp.broadcast_to(s4[:, h:h + 1], (_BLK, _HD)) for h in range(_HEADS)],
        axis=1) + 1e-16
    msg = msg / den
    h_ = jax.nn.gelu(jnp.dot(msg, wfc[...],
                             preferred_element_type=jnp.float32) + bfc[...])
    xn = x[...] + h_
    mu = jnp.mean(xn, axis=-1, keepdims=True)
    var = jnp.mean((xn - mu) * (xn - mu), axis=-1, keepdims=True)
    xn = (xn - mu) / jnp.sqrt(var + 1e-5) * g[...] + b[...]
    x_o[...] = xn
    if wq is not None:
        xq = jnp.dot(xn, wq[...], preferred_element_type=jnp.float32)
        xk = jnp.dot(xn, wk[...], preferred_element_type=jnp.float32)
        xv = jnp.dot(xn, wv[...], preferred_element_type=jnp.float32)
        xq_o[0] = xq[:, :64]
        xq_o[1] = xq[:, 64:]
        xk_o[0] = xk[:, :64]
        xk_o[1] = xk[:, 64:]
        xv_o[0] = xv[:, :64]
        xv_o[1] = xv[:, 64:]


def _merge(x, pm, ps, wfc, bfc, g, b, wq=None, wk=None, wv=None):
    full = lambda s: pl.BlockSpec(s, lambda i: (0, 0))
    out = jax.ShapeDtypeStruct((_N, _EMBED), jnp.float32)
    out2 = jax.ShapeDtypeStruct((2, _N, 64), jnp.float32)
    spec2 = pl.BlockSpec((2, _BLK, 64), lambda i: (0, i, 0))
    with_proj = wq is not None
    if with_proj:
        body = _merge_body
        args = (x, pm, ps, wfc, bfc, g, b, wq, wk, wv)
        w_specs = [full((_EMBED, _EMBED))] * 3
        out_shapes = [out, out2, out2, out2]
        n_out = 4
    else:
        body = lambda x, pm, ps, wfc, bfc, g, b, x_o: _merge_body(
            x, pm, ps, wfc, bfc, g, b, None, None, None, x_o, None, None, None)
        args = (x, pm, ps, wfc, bfc, g, b)
        w_specs = []
        out_shapes = [out]
        n_out = 1
    return pl.pallas_call(
        body,
        grid=(_N // _BLK,),
        in_specs=[
            pl.BlockSpec((_BLK, _EMBED), lambda i: (i, 0)),
            pl.BlockSpec((_NCORES, _BLK, 64), lambda i: (0, i, 0)),
            pl.BlockSpec((_NCORES, _BLK, 16), lambda i: (0, i, 0)),
            full((_EMBED, _EMBED)), full((1, _EMBED)),
            full((1, _EMBED)), full((1, _EMBED)),
        ] + w_specs,
        out_specs=[pl.BlockSpec((_BLK, _EMBED), lambda i: (i, 0))]
        + [spec2] * (n_out - 1),
        out_shape=out_shapes,
    )(*args)


def _dec_body(b8, x, w0, b0, w1, b1, wout, bout, out_o, gacc):
    i = pl.program_id(0)

    @pl.when(i == 0)
    def _():
        gacc[...] = jnp.zeros_like(gacc)

    bcol = b8[:, 0:1].astype(jnp.int32)
    oh = (lax.broadcasted_iota(jnp.int32, (_BLK, _EMBED), 1)
          == bcol).astype(jnp.float32)
    gacc[...] += lax.dot_general(oh, x[...], (((0,), (0,)), ((), ())),
                                 preferred_element_type=jnp.float32)

    @pl.when(i == pl.num_programs(0) - 1)
    def _():
        gemb = gacc[...]
        y = jax.nn.gelu(jnp.dot(gemb, w0[...],
                                preferred_element_type=jnp.float32) + b0[...])
        y = jax.nn.gelu(jnp.dot(y, w1[...],
                                preferred_element_type=jnp.float32) + b1[...])
        out_o[...] = jnp.dot(y, wout[...],
                             preferred_element_type=jnp.float32) + bout[...]


def _pool_decode(b8, x, w0, b0, w1, b1, wout, bout):
    full = lambda s: pl.BlockSpec(s, lambda i: (0, 0))
    odim = _OD * _NDP
    return pl.pallas_call(
        _dec_body,
        grid=(_N // _BLK,),
        in_specs=[
            pl.BlockSpec((_BLK, 8), lambda i: (i, 0)),
            pl.BlockSpec((_BLK, _EMBED), lambda i: (i, 0)),
            full((_EMBED, _EMBED)), full((1, _EMBED)),
            full((_EMBED, _EMBED)), full((1, _EMBED)),
            full((_EMBED, odim)), full((1, odim)),
        ],
        out_specs=pl.BlockSpec((_EMBED, odim), lambda i: (0, 0)),
        out_shape=jax.ShapeDtypeStruct((_EMBED, odim), jnp.float32),
        scratch_shapes=[pltpu.VMEM((_EMBED, _EMBED), jnp.float32)],
    )(b8, x, w0, b0, w1, b1, wout, bout)


@jax.jit
def _run(pos, params, z, edge_index, batch):
    src = edge_index[0]
    dst = edge_index[1]
    posT = jnp.zeros((3, _N), jnp.float32).at[:, :].set(pos.T)
    z8 = jnp.zeros((_N, 8), jnp.float32).at[:, 0].set(z.astype(jnp.float32))
    pos8 = jnp.zeros((_N, 8), jnp.float32).at[:, :3].set(pos)
    b8 = jnp.zeros((_N, 8), jnp.float32).at[:, 0].set(
        batch.astype(jnp.float32))

    embp = jnp.zeros((8, 32), jnp.float32).at[:5, :].set(params['emb'])
    w_init = params['W_init']
    wemb = w_init[:32]
    wpos = jnp.zeros((8, _EMBED), jnp.float32).at[:3, :].set(w_init[32:35])
    binit = params['b_init'].reshape(1, _EMBED)

    d2 = _edge_d2(posT, src, dst)
    d2c = d2.reshape(_E, 1)

    c0 = params['convs'][0]
    x, xq, xk, xv = _encoder(z8, pos8, embp, wemb, wpos, binit,
                             c0['Wq'], c0['Wk'], c0['Wv'])

    for li, c in enumerate(params['convs']):
        e = _e_proj(d2c, c['We'])
        pm, ps = _edge_attn(xq, xk, xv, e, src, dst)
        g = c['g'].reshape(1, _EMBED)
        b = c['b'].reshape(1, _EMBED)
        bfc = c['bfc'].reshape(1, _EMBED)
        if li + 1 < len(params['convs']):
            cn = params['convs'][li + 1]
            x, xq, xk, xv = _merge(x, pm, ps, c['Wfc'], bfc, g, b,
                                   cn['Wq'], cn['Wk'], cn['Wv'])
        else:
            (x,) = _merge(x, pm, ps, c['Wfc'], bfc, g, b)

    dec = params['dec']
    out = _pool_decode(b8, x, dec['W0'], dec['b0'].reshape(1, _EMBED),
                       dec['W1'], dec['b1'].reshape(1, _EMBED),
                       dec['Wout'], dec['bout'].reshape(1, _OD * _NDP))
    return out[:_G].reshape(_NDP * _G, _OD)


def kernel(pos, params, z, edge_index, batch):
    return _run(pos, params, z, edge_index, batch)


# speedup vs baseline: 8.8261x; 1.2851x over previous
import functools
import math

import jax
import jax.numpy as jnp
from jax import lax
from jax.experimental import pallas as pl
from jax.experimental.pallas import tpu as pltpu
from jax.experimental.pallas import tpu_sc as plsc

_N = 10000
_E = 320000
_G = 100
_EMBED = 128
_HEADS = 4
_HD = 32
_NR = 32
_CUTOFF = 6.0
_NDP = 512
_OD = 9

_NCORES = 2
_NSUB = 16
_NW = _NCORES * _NSUB
_EPW = _E // _NW
_BE = 128
_NBLK = 157
_EPT = _NBLK * _BE
_EPAD = _NSUB * _EPT
_NP = 10240
_NPT = _NP // _NSUB

_BLK = 2000
_EBLK = 2048


def _sc_mesh():
    return plsc.VectorSubcoreMesh(
        core_axis_name="c", subcore_axis_name="s",
        num_cores=_NCORES, num_subcores=_NSUB)


def _d2_body(posT_h, src_h, dst_h, out_h, pos_v, src_v, dst_v, out_v, sem):
    cid = lax.axis_index("c")
    sid = lax.axis_index("s")
    wid = cid * _NSUB + sid
    base = wid * _EPW
    pltpu.sync_copy(posT_h, pos_v)
    pltpu.sync_copy(src_h.at[pl.ds(base, _EPW)], src_v)
    pltpu.sync_copy(dst_h.at[pl.ds(base, _EPW)], dst_v)
    iota16 = lax.iota(jnp.int32, 16)
    zero = jnp.full((16,), 0, jnp.int32)
    one = jnp.full((16,), 1, jnp.int32)
    two = jnp.full((16,), 2, jnp.int32)

    def body(g, _):
        s_ids = src_v[pl.ds(g * 16, 16)]
        d_ids = dst_v[pl.ds(g * 16, 16)]
        dx = (plsc.load_gather(pos_v, [zero, s_ids])
              - plsc.load_gather(pos_v, [zero, d_ids]))
        dy = (plsc.load_gather(pos_v, [one, s_ids])
              - plsc.load_gather(pos_v, [one, d_ids]))
        dz = (plsc.load_gather(pos_v, [two, s_ids])
              - plsc.load_gather(pos_v, [two, d_ids]))
        out_v[pl.ds(g * 16, 16)] = dx * dx + dy * dy + dz * dz
        return 0

    lax.fori_loop(0, _EPW // 16, body, 0)
    pltpu.sync_copy(out_v, out_h.at[pl.ds(base, _EPW)])


def _edge_d2(posT, src, dst):
    kern = pl.kernel(
        _d2_body,
        out_type=jax.ShapeDtypeStruct((_E,), jnp.float32),
        mesh=_sc_mesh(),
        scratch_types=[
            pltpu.VMEM((3, _N), jnp.float32),
            pltpu.VMEM((_EPW,), jnp.int32),
            pltpu.VMEM((_EPW,), jnp.int32),
            pltpu.VMEM((_EPW,), jnp.float32),
            pltpu.SemaphoreType.DMA,
        ],
        compiler_params=pltpu.CompilerParams(needs_layout_passes=False, use_tc_tiling_on_sc=False),
    )
    return kern(posT, src, dst)


def _edge_body(xq_h, kv_h, e_h, idx_h, out_m,
               idx_v, q_v, kv_v, e_v, m_v, zb, msg_sh, sem):
    cid = lax.axis_index("c")
    sid = lax.axis_index("s")
    zero16 = jnp.zeros((16,), jnp.float32)

    def z1(i, _):
        zb[i // 5, pl.ds((i % 5) * 16, 16)] = zero16
        return 0
    lax.fori_loop(0, 128 * 5, z1, 0)

    def z3(i, _):
        m_v[i, pl.ds(64, 16)] = zero16
        return 0
    lax.fori_loop(0, _BE, z3, 0)

    for t in range(5):
        pltpu.sync_copy(zb, msg_sh.at[pl.ds(sid * _NPT + t * 128, 128)])
    plsc.subcore_barrier()

    inv_sqrt = 1.0 / math.sqrt(float(_HD))
    iota16 = lax.iota(jnp.int32, 16)

    def block(blk, _):
        pltpu.async_copy(idx_h.at[sid, blk], idx_v, sem).wait()
        sa = idx_v.at[0]
        da = idx_v.at[1]
        cps = [
            pltpu.async_copy(xq_h.at[cid].at[da], q_v, sem),
            pltpu.async_copy(kv_h.at[cid].at[sa], kv_v, sem),
            pltpu.async_copy(e_h.at[cid, pl.ds(sid * _EPT + blk * _BE, _BE)],
                             e_v, sem),
        ]
        for c in cps:
            c.wait()

        def group(g, _):
            eids = iota16 + g * 16
            for hh in range(2):
                acc = zero16
                for j in range(_HD):
                    col = jnp.full((16,), hh * _HD + j, jnp.int32)
                    qc = plsc.load_gather(q_v, [eids, col])
                    kc = plsc.load_gather(kv_v, [eids, col])
                    ec = plsc.load_gather(e_v, [eids, col])
                    acc = acc + qc * (kc + ec)
                w = jnp.exp(acc * inv_sqrt)
                lane = jnp.full((16,), 0, jnp.int32) + (64 + cid * 2 + hh)
                plsc.store_scatter(m_v, [eids, lane], w)
                for j in range(_HD):
                    colv = jnp.full((16,), 64 + hh * _HD + j, jnp.int32)
                    col = jnp.full((16,), hh * _HD + j, jnp.int32)
                    vc = plsc.load_gather(kv_v, [eids, colv])
                    ec = plsc.load_gather(e_v, [eids, col])
                    plsc.store_scatter(m_v, [eids, col], w * (vc + ec))
            return 0

        lax.fori_loop(0, _BE // 16, group, 0)
        pltpu.async_copy(m_v, msg_sh.at[da], sem, add=True).wait()
        return 0

    lax.fori_loop(0, _NBLK, block, 0)
    plsc.subcore_barrier()
    pltpu.sync_copy(msg_sh.at[pl.ds(sid * _NPT, _NPT)],
                    out_m.at[cid, pl.ds(sid * _NPT, _NPT)])


def _edge_attn(xq2, kv2, e2, idx_pack):
    kern = pl.kernel(
        _edge_body,
        out_type=jax.ShapeDtypeStruct((_NCORES, _NP, 80), jnp.float32),
        mesh=_sc_mesh(),
        scratch_types=[
            pltpu.VMEM((2, _BE), jnp.int32),
            pltpu.VMEM((_BE, 64), jnp.float32),
            pltpu.VMEM((_BE, _EMBED), jnp.float32),
            pltpu.VMEM((_BE, 64), jnp.float32),
            pltpu.VMEM((_BE, 80), jnp.float32),
            pltpu.VMEM((128, 80), jnp.float32),
            pltpu.VMEM_SHARED((_NP, 80), jnp.float32),
            pltpu.SemaphoreType.DMA,
        ],
        compiler_params=pltpu.CompilerParams(needs_layout_passes=False, use_tc_tiling_on_sc=False),
    )
    return kern(xq2, kv2, e2, idx_pack)


def _enc_body(z8, pos8, embp, wemb, wpos, binit, wq, wk, wv,
              x_o, xq_o, kv_o):
    zcol = z8[:, 0:1].astype(jnp.int32)
    ioh = lax.broadcasted_iota(jnp.int32, (_BLK, 8), 1)
    oh = (ioh == zcol).astype(jnp.float32)
    femb = jnp.dot(oh, embp[...], preferred_element_type=jnp.float32)
    x = (jnp.dot(femb, wemb[...], preferred_element_type=jnp.float32)
         + jnp.dot(pos8[...], wpos[...], preferred_element_type=jnp.float32)
         + binit[...])
    x_o[...] = x
    xq = jnp.dot(x, wq[...], preferred_element_type=jnp.float32)
    xk = jnp.dot(x, wk[...], preferred_element_type=jnp.float32)
    xv = jnp.dot(x, wv[...], preferred_element_type=jnp.float32)
    xq_o[0] = xq[:, :64]
    xq_o[1] = xq[:, 64:]
    kv_o[0] = jnp.concatenate([xk[:, :64], xv[:, :64]], axis=1)
    kv_o[1] = jnp.concatenate([xk[:, 64:], xv[:, 64:]], axis=1)


def _encoder(z8, pos8, embp, wemb, wpos, binit, wq, wk, wv):
    full = lambda s: pl.BlockSpec(s, lambda i: (0, 0))
    out = jax.ShapeDtypeStruct((_N, _EMBED), jnp.float32)
    outq = jax.ShapeDtypeStruct((2, _NP, 64), jnp.float32)
    outkv = jax.ShapeDtypeStruct((2, _NP, _EMBED), jnp.float32)
    specq = pl.BlockSpec((2, _BLK, 64), lambda i: (0, i, 0))
    speckv = pl.BlockSpec((2, _BLK, _EMBED), lambda i: (0, i, 0))
    return pl.pallas_call(
        _enc_body,
        grid=(_N // _BLK,),
        in_specs=[
            pl.BlockSpec((_BLK, 8), lambda i: (i, 0)),
            pl.BlockSpec((_BLK, 8), lambda i: (i, 0)),
            full((8, 32)), full((32, _EMBED)), full((8, _EMBED)),
            full((1, _EMBED)), full((_EMBED, _EMBED)),
            full((_EMBED, _EMBED)), full((_EMBED, _EMBED)),
        ],
        out_specs=[pl.BlockSpec((_BLK, _EMBED), lambda i: (i, 0)),
                   specq, speckv],
        out_shape=[out, outq, outkv],
    )(z8, pos8, embp, wemb, wpos, binit, wq, wk, wv)


def _e_body(d2, we, e_o):
    d = jnp.sqrt(d2[...] + 1e-12)
    width = _CUTOFF / _NR
    centers = lax.broadcasted_iota(jnp.int32, (1, _NR), 1).astype(
        jnp.float32) * (_CUTOFF / (_NR - 1))
    diff = d - centers
    rbf = jnp.exp(-(diff * diff) * (1.0 / (2.0 * width * width)))
    e = jnp.dot(rbf, we[...], preferred_element_type=jnp.float32)
    e_o[0] = e[:, :64]
    e_o[1] = e[:, 64:]


def _e_proj(d2c, we):
    return pl.pallas_call(
        _e_body,
        grid=(_EPAD // _EBLK,),
        in_specs=[
            pl.BlockSpec((_EBLK, 1), lambda i: (i, 0)),
            pl.BlockSpec((_NR, _EMBED), lambda i: (0, 0)),
        ],
        out_specs=pl.BlockSpec((2, _EBLK, 64), lambda i: (0, i, 0)),
        out_shape=jax.ShapeDtypeStruct((2, _EPAD, 64), jnp.float32),
    )(d2c, we)


def _merge_body(x, pm, wfc, bfc, g, b, wq, wk, wv,
                x_o, xq_o, kv_o):
    m = pm[...]
    msg = jnp.concatenate([m[0, :, :64], m[1, :, :64]], axis=1)
    s4 = m[0, :, 64:] + m[1, :, 64:]
    den = jnp.concatenate(
        [jnp.broadcast_to(s4[:, h:h + 1], (_BLK, _HD)) for h in range(_HEADS)],
        axis=1) + 1e-16
    msg = msg / den
    h_ = jax.nn.gelu(jnp.dot(msg, wfc[...],
                             preferred_element_type=jnp.float32) + bfc[...])
    xn = x[...] + h_
    mu = jnp.mean(xn, axis=-1, keepdims=True)
    var = jnp.mean((xn - mu) * (xn - mu), axis=-1, keepdims=True)
    xn = (xn - mu) / jnp.sqrt(var + 1e-5) * g[...] + b[...]
    x_o[...] = xn
    if wq is not None:
        xq = jnp.dot(xn, wq[...], preferred_element_type=jnp.float32)
        xk = jnp.dot(xn, wk[...], preferred_element_type=jnp.float32)
        xv = jnp.dot(xn, wv[...], preferred_element_type=jnp.float32)
        xq_o[0] = xq[:, :64]
        xq_o[1] = xq[:, 64:]
        kv_o[0] = jnp.concatenate([xk[:, :64], xv[:, :64]], axis=1)
        kv_o[1] = jnp.concatenate([xk[:, 64:], xv[:, 64:]], axis=1)


def _merge(x, pm, wfc, bfc, g, b, wq=None, wk=None, wv=None):
    full = lambda s: pl.BlockSpec(s, lambda i: (0, 0))
    out = jax.ShapeDtypeStruct((_N, _EMBED), jnp.float32)
    outq = jax.ShapeDtypeStruct((2, _NP, 64), jnp.float32)
    outkv = jax.ShapeDtypeStruct((2, _NP, _EMBED), jnp.float32)
    specq = pl.BlockSpec((2, _BLK, 64), lambda i: (0, i, 0))
    speckv = pl.BlockSpec((2, _BLK, _EMBED), lambda i: (0, i, 0))
    with_proj = wq is not None
    if with_proj:
        body = _merge_body
        args = (x, pm, wfc, bfc, g, b, wq, wk, wv)
        w_specs = [full((_EMBED, _EMBED))] * 3
        out_shapes = [out, outq, outkv]
        o_specs = [pl.BlockSpec((_BLK, _EMBED), lambda i: (i, 0)),
                   specq, speckv]
    else:
        body = lambda x, pm, wfc, bfc, g, b, x_o: _merge_body(
            x, pm, wfc, bfc, g, b, None, None, None, x_o, None, None)
        args = (x, pm, wfc, bfc, g, b)
        w_specs = []
        out_shapes = [out]
        o_specs = [pl.BlockSpec((_BLK, _EMBED), lambda i: (i, 0))]
    return pl.pallas_call(
        body,
        grid=(_N // _BLK,),
        in_specs=[
            pl.BlockSpec((_BLK, _EMBED), lambda i: (i, 0)),
            pl.BlockSpec((_NCORES, _BLK, 80), lambda i: (0, i, 0)),
            full((_EMBED, _EMBED)), full((1, _EMBED)),
            full((1, _EMBED)), full((1, _EMBED)),
        ] + w_specs,
        out_specs=o_specs,
        out_shape=out_shapes,
    )(*args)


def _dec_body(b8, x, w0, b0, w1, b1, wout, bout, out_o, gacc):
    i = pl.program_id(0)

    @pl.when(i == 0)
    def _():
        gacc[...] = jnp.zeros_like(gacc)

    bcol = b8[:, 0:1].astype(jnp.int32)
    oh = (lax.broadcasted_iota(jnp.int32, (_BLK, _EMBED), 1)
          == bcol).astype(jnp.float32)
    gacc[...] += lax.dot_general(oh, x[...], (((0,), (0,)), ((), ())),
                                 preferred_element_type=jnp.float32)

    @pl.when(i == pl.num_programs(0) - 1)
    def _():
        gemb = gacc[...]
        y = jax.nn.gelu(jnp.dot(gemb, w0[...],
                                preferred_element_type=jnp.float32) + b0[...])
        y = jax.nn.gelu(jnp.dot(y, w1[...],
                                preferred_element_type=jnp.float32) + b1[...])
        out_o[...] = jnp.dot(y, wout[...],
                             preferred_element_type=jnp.float32) + bout[...]


def _pool_decode(b8, x, w0, b0, w1, b1, wout, bout):
    full = lambda s: pl.BlockSpec(s, lambda i: (0, 0))
    odim = _OD * _NDP
    return pl.pallas_call(
        _dec_body,
        grid=(_N // _BLK,),
        in_specs=[
            pl.BlockSpec((_BLK, 8), lambda i: (i, 0)),
            pl.BlockSpec((_BLK, _EMBED), lambda i: (i, 0)),
            full((_EMBED, _EMBED)), full((1, _EMBED)),
            full((_EMBED, _EMBED)), full((1, _EMBED)),
            full((_EMBED, odim)), full((1, odim)),
        ],
        out_specs=pl.BlockSpec((_EMBED, odim), lambda i: (0, 0)),
        out_shape=jax.ShapeDtypeStruct((_EMBED, odim), jnp.float32),
        scratch_shapes=[pltpu.VMEM((_EMBED, _EMBED), jnp.float32)],
    )(b8, x, w0, b0, w1, b1, wout, bout)


@jax.jit
def _run(pos, params, z, edge_index, batch):
    src = edge_index[0]
    dst = edge_index[1]
    posT = jnp.zeros((3, _N), jnp.float32).at[:, :].set(pos.T)
    z8 = jnp.zeros((_N, 8), jnp.float32).at[:, 0].set(z.astype(jnp.float32))
    pos8 = jnp.zeros((_N, 8), jnp.float32).at[:, :3].set(pos)
    b8 = jnp.zeros((_N, 8), jnp.float32).at[:, 0].set(
        batch.astype(jnp.float32))

    embp = jnp.zeros((8, 32), jnp.float32).at[:5, :].set(params['emb'])
    w_init = params['W_init']
    wemb = w_init[:32]
    wpos = jnp.zeros((8, _EMBED), jnp.float32).at[:3, :].set(w_init[32:35])
    binit = params['b_init'].reshape(1, _EMBED)

    npad = _EPAD - _E
    src_p = jnp.concatenate([src, jnp.zeros((npad,), src.dtype)])
    dst_p = jnp.concatenate([dst, jnp.full((npad,), _N, dst.dtype)])
    idx_pack = jnp.stack([src_p.reshape(_NSUB, _NBLK, _BE),
                          dst_p.reshape(_NSUB, _NBLK, _BE)], axis=2)

    d2 = _edge_d2(posT, src, dst)
    d2c = jnp.concatenate([d2, jnp.zeros((npad,), jnp.float32)]).reshape(
        _EPAD, 1)

    c0 = params['convs'][0]
    x, xq, kv = _encoder(z8, pos8, embp, wemb, wpos, binit,
                         c0['Wq'], c0['Wk'], c0['Wv'])

    for li, c in enumerate(params['convs']):
        e = _e_proj(d2c, c['We'])
        pm = _edge_attn(xq, kv, e, idx_pack)
        g = c['g'].reshape(1, _EMBED)
        b = c['b'].reshape(1, _EMBED)
        bfc = c['bfc'].reshape(1, _EMBED)
        if li + 1 < len(params['convs']):
            cn = params['convs'][li + 1]
            x, xq, kv = _merge(x, pm, c['Wfc'], bfc, g, b,
                               cn['Wq'], cn['Wk'], cn['Wv'])
        else:
            (x,) = _merge(x, pm, c['Wfc'], bfc, g, b)

    dec = params['dec']
    out = _pool_decode(b8, x, dec['W0'], dec['b0'].reshape(1, _EMBED),
                       dec['W1'], dec['b1'].reshape(1, _EMBED),
                       dec['Wout'], dec['bout'].reshape(1, _OD * _NDP))
    return out[:_G].reshape(_NDP * _G, _OD)


def kernel(pos, params, z, edge_index, batch):
    return _run(pos, params, z, edge_index, batch)


# speedup vs baseline: 26.8580x; 3.0430x over previous
import functools
import math

import jax
import jax.numpy as jnp
from jax import lax
from jax.experimental import pallas as pl
from jax.experimental.pallas import tpu as pltpu
from jax.experimental.pallas import tpu_sc as plsc

_N = 10000
_E = 320000
_G = 100
_EMBED = 128
_HEADS = 4
_HD = 32
_NR = 32
_CUTOFF = 6.0
_NDP = 512
_OD = 9

_NCORES = 2
_NSUB = 16
_NW = _NCORES * _NSUB
_EPW = _E // _NW
_BE = 128
_NBLK = 157
_EPT = _NBLK * _BE
_EPAD = _NSUB * _EPT
_NP = 10240
_NPT = _NP // _NSUB

_BLK = 2000
_EBLK = 2048


def _sc_mesh():
    return plsc.VectorSubcoreMesh(
        core_axis_name="c", subcore_axis_name="s",
        num_cores=_NCORES, num_subcores=_NSUB)


def _d2_body(posT_h, src_h, dst_h, out_h, pos_v, src_v, dst_v, out_v, sem):
    cid = lax.axis_index("c")
    sid = lax.axis_index("s")
    wid = cid * _NSUB + sid
    base = wid * _EPW
    pltpu.sync_copy(posT_h, pos_v)
    pltpu.sync_copy(src_h.at[pl.ds(base, _EPW)], src_v)
    pltpu.sync_copy(dst_h.at[pl.ds(base, _EPW)], dst_v)
    iota16 = lax.iota(jnp.int32, 16)
    zero = jnp.full((16,), 0, jnp.int32)
    one = jnp.full((16,), 1, jnp.int32)
    two = jnp.full((16,), 2, jnp.int32)

    def body(g, _):
        s_ids = src_v[pl.ds(g * 16, 16)]
        d_ids = dst_v[pl.ds(g * 16, 16)]
        dx = (plsc.load_gather(pos_v, [zero, s_ids])
              - plsc.load_gather(pos_v, [zero, d_ids]))
        dy = (plsc.load_gather(pos_v, [one, s_ids])
              - plsc.load_gather(pos_v, [one, d_ids]))
        dz = (plsc.load_gather(pos_v, [two, s_ids])
              - plsc.load_gather(pos_v, [two, d_ids]))
        out_v[pl.ds(g * 16, 16)] = dx * dx + dy * dy + dz * dz
        return 0

    lax.fori_loop(0, _EPW // 16, body, 0)
    pltpu.sync_copy(out_v, out_h.at[pl.ds(base, _EPW)])


def _edge_d2(posT, src, dst):
    kern = pl.kernel(
        _d2_body,
        out_type=jax.ShapeDtypeStruct((_E,), jnp.float32),
        mesh=_sc_mesh(),
        scratch_types=[
            pltpu.VMEM((3, _N), jnp.float32),
            pltpu.VMEM((_EPW,), jnp.int32),
            pltpu.VMEM((_EPW,), jnp.int32),
            pltpu.VMEM((_EPW,), jnp.float32),
            pltpu.SemaphoreType.DMA,
        ],
        compiler_params=pltpu.CompilerParams(needs_layout_passes=False, use_tc_tiling_on_sc=False),
    )
    return kern(posT, src, dst)


def _edge_body(xq_h, kv_h, e_h, idx_h, out_m,
               idx_v, q_v, kv_v, e_v, m_v, zb, msg_sh, sem):
    cid = lax.axis_index("c")
    sid = lax.axis_index("s")
    zero16 = jnp.zeros((16,), jnp.float32)

    def z1(i, _):
        zb[i // 5, pl.ds((i % 5) * 16, 16)] = zero16
        return 0
    lax.fori_loop(0, 128 * 5, z1, 0)

    for t in range(5):
        pltpu.sync_copy(zb, msg_sh.at[pl.ds(sid * _NPT + t * 128, 128)])
    plsc.subcore_barrier()

    inv_sqrt = 1.0 / math.sqrt(float(_HD))
    iota16 = lax.iota(jnp.int32, 16)

    def block(blk, _):
        pltpu.async_copy(idx_h.at[sid, blk], idx_v, sem).wait()
        sa = idx_v.at[0]
        da = idx_v.at[1]
        cps = [
            pltpu.async_copy(xq_h.at[cid].at[da], q_v, sem),
            pltpu.async_copy(kv_h.at[cid].at[sa], kv_v, sem),
            pltpu.async_copy(e_h.at[cid, pl.ds(sid * _EPT + blk * _BE, _BE)],
                             e_v, sem),
        ]
        for c in cps:
            c.wait()

        def edge(i, _):
            q = [q_v[i, pl.ds(16 * t, 16)] for t in range(4)]
            k = [kv_v[i, pl.ds(16 * t, 16)] for t in range(4)]
            v = [kv_v[i, pl.ds(64 + 16 * t, 16)] for t in range(4)]
            ee = [e_v[i, pl.ds(16 * t, 16)] for t in range(4)]
            p = [q[t] * (k[t] + ee[t]) for t in range(4)]
            s0 = jnp.sum(p[0] + p[1])
            s1 = jnp.sum(p[2] + p[3])
            w0 = jnp.exp(jnp.full((16,), s0, jnp.float32) * inv_sqrt)
            w1 = jnp.exp(jnp.full((16,), s1, jnp.float32) * inv_sqrt)
            m_v[i, pl.ds(0, 16)] = w0 * (v[0] + ee[0])
            m_v[i, pl.ds(16, 16)] = w0 * (v[1] + ee[1])
            m_v[i, pl.ds(32, 16)] = w1 * (v[2] + ee[2])
            m_v[i, pl.ds(48, 16)] = w1 * (v[3] + ee[3])
            wl = (jnp.where(iota16 == cid * 2, w0, zero16)
                  + jnp.where(iota16 == cid * 2 + 1, w1, zero16))
            m_v[i, pl.ds(64, 16)] = wl
            return 0

        lax.fori_loop(0, _BE, edge, 0)
        pltpu.async_copy(m_v, msg_sh.at[da], sem, add=True).wait()
        return 0

    lax.fori_loop(0, _NBLK, block, 0)
    plsc.subcore_barrier()
    pltpu.sync_copy(msg_sh.at[pl.ds(sid * _NPT, _NPT)],
                    out_m.at[cid, pl.ds(sid * _NPT, _NPT)])


def _edge_attn(xq2, kv2, e2, idx_pack):
    kern = pl.kernel(
        _edge_body,
        out_type=jax.ShapeDtypeStruct((_NCORES, _NP, 80), jnp.float32),
        mesh=_sc_mesh(),
        scratch_types=[
            pltpu.VMEM((2, _BE), jnp.int32),
            pltpu.VMEM((_BE, 64), jnp.float32),
            pltpu.VMEM((_BE, _EMBED), jnp.float32),
            pltpu.VMEM((_BE, 64), jnp.float32),
            pltpu.VMEM((_BE, 80), jnp.float32),
            pltpu.VMEM((128, 80), jnp.float32),
            pltpu.VMEM_SHARED((_NP, 80), jnp.float32),
            pltpu.SemaphoreType.DMA,
        ],
        compiler_params=pltpu.CompilerParams(needs_layout_passes=False, use_tc_tiling_on_sc=False),
    )
    return kern(xq2, kv2, e2, idx_pack)


def _enc_body(z8, pos8, embp, wemb, wpos, binit, wq, wk, wv,
              x_o, xq_o, kv_o):
    zcol = z8[:, 0:1].astype(jnp.int32)
    ioh = lax.broadcasted_iota(jnp.int32, (_BLK, 8), 1)
    oh = (ioh == zcol).astype(jnp.float32)
    femb = jnp.dot(oh, embp[...], preferred_element_type=jnp.float32)
    x = (jnp.dot(femb, wemb[...], preferred_element_type=jnp.float32)
         + jnp.dot(pos8[...], wpos[...], preferred_element_type=jnp.float32)
         + binit[...])
    x_o[...] = x
    xq = jnp.dot(x, wq[...], preferred_element_type=jnp.float32)
    xk = jnp.dot(x, wk[...], preferred_element_type=jnp.float32)
    xv = jnp.dot(x, wv[...], preferred_element_type=jnp.float32)
    xq_o[0] = xq[:, :64]
    xq_o[1] = xq[:, 64:]
    kv_o[0] = jnp.concatenate([xk[:, :64], xv[:, :64]], axis=1)
    kv_o[1] = jnp.concatenate([xk[:, 64:], xv[:, 64:]], axis=1)


def _encoder(z8, pos8, embp, wemb, wpos, binit, wq, wk, wv):
    full = lambda s: pl.BlockSpec(s, lambda i: (0, 0))
    out = jax.ShapeDtypeStruct((_N, _EMBED), jnp.float32)
    outq = jax.ShapeDtypeStruct((2, _NP, 64), jnp.float32)
    outkv = jax.ShapeDtypeStruct((2, _NP, _EMBED), jnp.float32)
    specq = pl.BlockSpec((2, _BLK, 64), lambda i: (0, i, 0))
    speckv = pl.BlockSpec((2, _BLK, _EMBED), lambda i: (0, i, 0))
    return pl.pallas_call(
        _enc_body,
        grid=(_N // _BLK,),
        in_specs=[
            pl.BlockSpec((_BLK, 8), lambda i: (i, 0)),
            pl.BlockSpec((_BLK, 8), lambda i: (i, 0)),
            full((8, 32)), full((32, _EMBED)), full((8, _EMBED)),
            full((1, _EMBED)), full((_EMBED, _EMBED)),
            full((_EMBED, _EMBED)), full((_EMBED, _EMBED)),
        ],
        out_specs=[pl.BlockSpec((_BLK, _EMBED), lambda i: (i, 0)),
                   specq, speckv],
        out_shape=[out, outq, outkv],
    )(z8, pos8, embp, wemb, wpos, binit, wq, wk, wv)


def _e_body(d2, we, e_o):
    d = jnp.sqrt(d2[...] + 1e-12)
    width = _CUTOFF / _NR
    centers = lax.broadcasted_iota(jnp.int32, (1, _NR), 1).astype(
        jnp.float32) * (_CUTOFF / (_NR - 1))
    diff = d - centers
    rbf = jnp.exp(-(diff * diff) * (1.0 / (2.0 * width * width)))
    e = jnp.dot(rbf, we[...], preferred_element_type=jnp.float32)
    e_o[0] = e[:, :64]
    e_o[1] = e[:, 64:]


def _e_proj(d2c, we):
    return pl.pallas_call(
        _e_body,
        grid=(_EPAD // _EBLK,),
        in_specs=[
            pl.BlockSpec((_EBLK, 1), lambda i: (i, 0)),
            pl.BlockSpec((_NR, _EMBED), lambda i: (0, 0)),
        ],
        out_specs=pl.BlockSpec((2, _EBLK, 64), lambda i: (0, i, 0)),
        out_shape=jax.ShapeDtypeStruct((2, _EPAD, 64), jnp.float32),
    )(d2c, we)


def _merge_body(x, pm, wfc, bfc, g, b, wq, wk, wv,
                x_o, xq_o, kv_o):
    m = pm[...]
    msg = jnp.concatenate([m[0, :, :64], m[1, :, :64]], axis=1)
    s4 = m[0, :, 64:] + m[1, :, 64:]
    den = jnp.concatenate(
        [jnp.broadcast_to(s4[:, h:h + 1], (_BLK, _HD)) for h in range(_HEADS)],
        axis=1) + 1e-16
    msg = msg / den
    h_ = jax.nn.gelu(jnp.dot(msg, wfc[...],
                             preferred_element_type=jnp.float32) + bfc[...])
    xn = x[...] + h_
    mu = jnp.mean(xn, axis=-1, keepdims=True)
    var = jnp.mean((xn - mu) * (xn - mu), axis=-1, keepdims=True)
    xn = (xn - mu) / jnp.sqrt(var + 1e-5) * g[...] + b[...]
    x_o[...] = xn
    if wq is not None:
        xq = jnp.dot(xn, wq[...], preferred_element_type=jnp.float32)
        xk = jnp.dot(xn, wk[...], preferred_element_type=jnp.float32)
        xv = jnp.dot(xn, wv[...], preferred_element_type=jnp.float32)
        xq_o[0] = xq[:, :64]
        xq_o[1] = xq[:, 64:]
        kv_o[0] = jnp.concatenate([xk[:, :64], xv[:, :64]], axis=1)
        kv_o[1] = jnp.concatenate([xk[:, 64:], xv[:, 64:]], axis=1)


def _merge(x, pm, wfc, bfc, g, b, wq=None, wk=None, wv=None):
    full = lambda s: pl.BlockSpec(s, lambda i: (0, 0))
    out = jax.ShapeDtypeStruct((_N, _EMBED), jnp.float32)
    outq = jax.ShapeDtypeStruct((2, _NP, 64), jnp.float32)
    outkv = jax.ShapeDtypeStruct((2, _NP, _EMBED), jnp.float32)
    specq = pl.BlockSpec((2, _BLK, 64), lambda i: (0, i, 0))
    speckv = pl.BlockSpec((2, _BLK, _EMBED), lambda i: (0, i, 0))
    with_proj = wq is not None
    if with_proj:
        body = _merge_body
        args = (x, pm, wfc, bfc, g, b, wq, wk, wv)
        w_specs = [full((_EMBED, _EMBED))] * 3
        out_shapes = [out, outq, outkv]
        o_specs = [pl.BlockSpec((_BLK, _EMBED), lambda i: (i, 0)),
                   specq, speckv]
    else:
        body = lambda x, pm, wfc, bfc, g, b, x_o: _merge_body(
            x, pm, wfc, bfc, g, b, None, None, None, x_o, None, None)
        args = (x, pm, wfc, bfc, g, b)
        w_specs = []
        out_shapes = [out]
        o_specs = [pl.BlockSpec((_BLK, _EMBED), lambda i: (i, 0))]
    return pl.pallas_call(
        body,
        grid=(_N // _BLK,),
        in_specs=[
            pl.BlockSpec((_BLK, _EMBED), lambda i: (i, 0)),
            pl.BlockSpec((_NCORES, _BLK, 80), lambda i: (0, i, 0)),
            full((_EMBED, _EMBED)), full((1, _EMBED)),
            full((1, _EMBED)), full((1, _EMBED)),
        ] + w_specs,
        out_specs=o_specs,
        out_shape=out_shapes,
    )(*args)


def _dec_body(b8, x, w0, b0, w1, b1, wout, bout, out_o, gacc):
    i = pl.program_id(0)

    @pl.when(i == 0)
    def _():
        gacc[...] = jnp.zeros_like(gacc)

    bcol = b8[:, 0:1].astype(jnp.int32)
    oh = (lax.broadcasted_iota(jnp.int32, (_BLK, _EMBED), 1)
          == bcol).astype(jnp.float32)
    gacc[...] += lax.dot_general(oh, x[...], (((0,), (0,)), ((), ())),
                                 preferred_element_type=jnp.float32)

    @pl.when(i == pl.num_programs(0) - 1)
    def _():
        gemb = gacc[...]
        y = jax.nn.gelu(jnp.dot(gemb, w0[...],
                                preferred_element_type=jnp.float32) + b0[...])
        y = jax.nn.gelu(jnp.dot(y, w1[...],
                                preferred_element_type=jnp.float32) + b1[...])
        out_o[...] = jnp.dot(y, wout[...],
                             preferred_element_type=jnp.float32) + bout[...]


def _pool_decode(b8, x, w0, b0, w1, b1, wout, bout):
    full = lambda s: pl.BlockSpec(s, lambda i: (0, 0))
    odim = _OD * _NDP
    return pl.pallas_call(
        _dec_body,
        grid=(_N // _BLK,),
        in_specs=[
            pl.BlockSpec((_BLK, 8), lambda i: (i, 0)),
            pl.BlockSpec((_BLK, _EMBED), lambda i: (i, 0)),
            full((_EMBED, _EMBED)), full((1, _EMBED)),
            full((_EMBED, _EMBED)), full((1, _EMBED)),
            full((_EMBED, odim)), full((1, odim)),
        ],
        out_specs=pl.BlockSpec((_EMBED, odim), lambda i: (0, 0)),
        out_shape=jax.ShapeDtypeStruct((_EMBED, odim), jnp.float32),
        scratch_shapes=[pltpu.VMEM((_EMBED, _EMBED), jnp.float32)],
    )(b8, x, w0, b0, w1, b1, wout, bout)


@jax.jit
def _run(pos, params, z, edge_index, batch):
    src = edge_index[0]
    dst = edge_index[1]
    posT = jnp.zeros((3, _N), jnp.float32).at[:, :].set(pos.T)
    z8 = jnp.zeros((_N, 8), jnp.float32).at[:, 0].set(z.astype(jnp.float32))
    pos8 = jnp.zeros((_N, 8), jnp.float32).at[:, :3].set(pos)
    b8 = jnp.zeros((_N, 8), jnp.float32).at[:, 0].set(
        batch.astype(jnp.float32))

    embp = jnp.zeros((8, 32), jnp.float32).at[:5, :].set(params['emb'])
    w_init = params['W_init']
    wemb = w_init[:32]
    wpos = jnp.zeros((8, _EMBED), jnp.float32).at[:3, :].set(w_init[32:35])
    binit = params['b_init'].reshape(1, _EMBED)

    npad = _EPAD - _E
    src_p = jnp.concatenate([src, jnp.zeros((npad,), src.dtype)])
    dst_p = jnp.concatenate([dst, jnp.full((npad,), _N, dst.dtype)])
    idx_pack = jnp.stack([src_p.reshape(_NSUB, _NBLK, _BE),
                          dst_p.reshape(_NSUB, _NBLK, _BE)], axis=2)

    d2 = _edge_d2(posT, src, dst)
    d2c = jnp.concatenate([d2, jnp.zeros((npad,), jnp.float32)]).reshape(
        _EPAD, 1)

    c0 = params['convs'][0]
    x, xq, kv = _encoder(z8, pos8, embp, wemb, wpos, binit,
                         c0['Wq'], c0['Wk'], c0['Wv'])

    for li, c in enumerate(params['convs']):
        e = _e_proj(d2c, c['We'])
        pm = _edge_attn(xq, kv, e, idx_pack)
        g = c['g'].reshape(1, _EMBED)
        b = c['b'].reshape(1, _EMBED)
        bfc = c['bfc'].reshape(1, _EMBED)
        if li + 1 < len(params['convs']):
            cn = params['convs'][li + 1]
            x, xq, kv = _merge(x, pm, c['Wfc'], bfc, g, b,
                               cn['Wq'], cn['Wk'], cn['Wv'])
        else:
            (x,) = _merge(x, pm, c['Wfc'], bfc, g, b)

    dec = params['dec']
    out = _pool_decode(b8, x, dec['W0'], dec['b0'].reshape(1, _EMBED),
                       dec['W1'], dec['b1'].reshape(1, _EMBED),
                       dec['Wout'], dec['bout'].reshape(1, _OD * _NDP))
    return out[:_G].reshape(_NDP * _G, _OD)


def kernel(pos, params, z, edge_index, batch):
    return _run(pos, params, z, edge_index, batch)
